# baseline (speedup 1.0000x reference)
import functools
import jax
import jax.numpy as jnp
from jax import lax
from jax.experimental import pallas as pl
from jax.experimental.pallas import tpu as pltpu, tpu_sc as plsc

SEQ = 4608
D = 768
BATCH = 4
NC = 2
NS = 16
NW = NC * NS
ROWS_W = SEQ // NW
CH = 16
NCH = ROWS_W // CH
LANE = 16
DVEC = D // LANE
NPAIR = BATCH // 2
NT = NCH * NPAIR
NXB = 6


def _body(x_hbm, emb_hbm, out_hbm,
          xb0, xb1, xb2, xb3, xb4, xb5, eb0, eb1,
          sx0, sx1, sx2, sx3, sx4, sx5, se0, se1,
          so0, so1, so2, so3, so4, so5):
    xbuf = [xb0, xb1, xb2, xb3, xb4, xb5]
    ebuf = [eb0, eb1]
    sem_x = [sx0, sx1, sx2, sx3, sx4, sx5]
    sem_e = [se0, se1]
    sem_o = [so0, so1, so2, so3, so4, so5]

    wid = lax.axis_index("s") * NC + lax.axis_index("c")
    base = wid * ROWS_W

    e_desc = [None] * NCH
    x_desc = [None] * (2 * NT)
    o_desc = [None] * (2 * NT)

    e_desc[0] = pltpu.async_copy(
        emb_hbm.at[pl.ds(base, CH)], ebuf[0], sem_e[0])
    for i in range(2):
        x_desc[i] = pltpu.async_copy(
            x_hbm.at[i, pl.ds(base, CH)], xbuf[i], sem_x[i])

    for t in range(NT):
        c, p = divmod(t, NPAIR)
        s0 = (2 * t) % NXB
        s1 = (2 * t + 1) % NXB
        xa = xbuf[s0]
        xc = xbuf[s1]
        eb = ebuf[c % 2]
        row0 = base + c * CH

        if t + 1 < NT:
            c2, p2 = divmod(t + 1, NPAIR)
            row2 = base + c2 * CH
            if p2 == 0:
                e_desc[c2] = pltpu.async_copy(
                    emb_hbm.at[pl.ds(row2, CH)], ebuf[c2 % 2], sem_e[c2 % 2])
            if t - 2 >= 0:
                o_desc[2 * (t - 2)].wait()
                o_desc[2 * (t - 2) + 1].wait()
            x_desc[2 * (t + 1)] = pltpu.async_copy(
                x_hbm.at[2 * p2, pl.ds(row2, CH)],
                xbuf[(2 * t + 2) % NXB], sem_x[(2 * t + 2) % NXB])
            x_desc[2 * (t + 1) + 1] = pltpu.async_copy(
                x_hbm.at[2 * p2 + 1, pl.ds(row2, CH)],
                xbuf[(2 * t + 3) % NXB], sem_x[(2 * t + 3) % NXB])

        x_desc[2 * t].wait()
        x_desc[2 * t + 1].wait()
        if p == 0:
            e_desc[c].wait()

        def add_row(r, _, xa=xa, xc=xc, eb=eb):
            for j in range(DVEC):
                sl = pl.ds(j * LANE, LANE)
                e = eb[r, sl]
                xa[r, sl] = xa[r, sl] + e
                xc[r, sl] = xc[r, sl] + e
            return 0

        lax.fori_loop(0, CH, add_row, 0)

        o_desc[2 * t] = pltpu.async_copy(
            xa, out_hbm.at[2 * p, pl.ds(row0, CH)], sem_o[s0])
        o_desc[2 * t + 1] = pltpu.async_copy(
            xc, out_hbm.at[2 * p + 1, pl.ds(row0, CH)], sem_o[s1])

    for t in range(2 * NT - 6, 2 * NT):
        o_desc[t].wait()


def kernel(x, emb):
    mesh = plsc.VectorSubcoreMesh(core_axis_name="c", subcore_axis_name="s")
    k = functools.partial(
        pl.kernel,
        mesh=mesh,
        out_type=jax.ShapeDtypeStruct((BATCH, SEQ, D), jnp.float32),
        scratch_types=(
            [pltpu.VMEM((CH, D), jnp.float32)] * 8
            + [pltpu.SemaphoreType.DMA] * 14
        ),
    )(_body)
    return k(x, emb)


# speedup vs baseline: 1.1129x; 1.1129x over previous
import functools
import jax
import jax.numpy as jnp
from jax import lax
from jax.experimental import pallas as pl
from jax.experimental.pallas import tpu as pltpu, tpu_sc as plsc

SEQ = 4608
D = 768
BATCH = 4
NC = 2
NS = 16
NW = NC * NS
ROWS_W = SEQ // NW
CH = 24
NCH = ROWS_W // CH
LANE = 16
DVEC = D // LANE
NPAIR = BATCH // 2
NT = NCH * NPAIR


def _body(x_hbm, emb_hbm, out_hbm,
          xb0, xb1, xb2, xb3, eb0, eb1,
          sx0, sx1, sx2, sx3, se0, se1, so0, so1, so2, so3):
    xbuf = [xb0, xb1, xb2, xb3]
    ebuf = [eb0, eb1]
    sem_x = [sx0, sx1, sx2, sx3]
    sem_e = [se0, se1]
    sem_o = [so0, so1, so2, so3]

    wid = lax.axis_index("s") * NC + lax.axis_index("c")
    base = wid * ROWS_W

    e_desc = [None] * NCH
    x_desc = [None] * (2 * NT)
    o_desc = [None] * (2 * NT)

    e_desc[0] = pltpu.async_copy(
        emb_hbm.at[pl.ds(base, CH)], ebuf[0], sem_e[0])
    x_desc[0] = pltpu.async_copy(
        x_hbm.at[0, pl.ds(base, CH)], xbuf[0], sem_x[0])
    x_desc[1] = pltpu.async_copy(
        x_hbm.at[1, pl.ds(base, CH)], xbuf[1], sem_x[1])

    for t in range(NT):
        c, p = divmod(t, NPAIR)
        s0 = (2 * t) % 4
        s1 = (2 * t + 1) % 4
        xa = xbuf[s0]
        xc = xbuf[s1]
        eb = ebuf[c % 2]
        row0 = base + c * CH

        if t + 1 < NT:
            c2, p2 = divmod(t + 1, NPAIR)
            row2 = base + c2 * CH
            if p2 == 0:
                e_desc[c2] = pltpu.async_copy(
                    emb_hbm.at[pl.ds(row2, CH)], ebuf[c2 % 2], sem_e[c2 % 2])
            if t - 1 >= 0:
                o_desc[2 * (t - 1)].wait()
                o_desc[2 * (t - 1) + 1].wait()
            x_desc[2 * (t + 1)] = pltpu.async_copy(
                x_hbm.at[2 * p2, pl.ds(row2, CH)],
                xbuf[(2 * t + 2) % 4], sem_x[(2 * t + 2) % 4])
            x_desc[2 * (t + 1) + 1] = pltpu.async_copy(
                x_hbm.at[2 * p2 + 1, pl.ds(row2, CH)],
                xbuf[(2 * t + 3) % 4], sem_x[(2 * t + 3) % 4])

        x_desc[2 * t].wait()
        x_desc[2 * t + 1].wait()
        if p == 0:
            e_desc[c].wait()

        def add_row(r, _, xa=xa, xc=xc, eb=eb):
            for j in range(DVEC):
                sl = pl.ds(j * LANE, LANE)
                e = eb[r, sl]
                xa[r, sl] = xa[r, sl] + e
                xc[r, sl] = xc[r, sl] + e
            return 0

        lax.fori_loop(0, CH, add_row, 0)

        o_desc[2 * t] = pltpu.async_copy(
            xa, out_hbm.at[2 * p, pl.ds(row0, CH)], sem_o[s0])
        o_desc[2 * t + 1] = pltpu.async_copy(
            xc, out_hbm.at[2 * p + 1, pl.ds(row0, CH)], sem_o[s1])

    for t in range(2 * NT - 4, 2 * NT):
        o_desc[t].wait()


def kernel(x, emb):
    mesh = plsc.VectorSubcoreMesh(core_axis_name="c", subcore_axis_name="s")
    k = functools.partial(
        pl.kernel,
        mesh=mesh,
        out_type=jax.ShapeDtypeStruct((BATCH, SEQ, D), jnp.float32),
        scratch_types=(
            [pltpu.VMEM((CH, D), jnp.float32)] * 6
            + [pltpu.SemaphoreType.DMA] * 10
        ),
    )(_body)
    return k(x, emb)
